# baseline (speedup 1.0000x reference)
import functools

import jax
import jax.numpy as jnp
from jax import lax
from jax.experimental import pallas as pl
from jax.experimental.pallas import tpu as pltpu
from jax.experimental.pallas import tpu_sc as plsc

NUM_AXIOMS = 4096
D_AXIOM = 449
D_PAD = 464
D_TILE = 456
BATCH = 16384

_NUM_CORES = 2
_NUM_SUBCORES = 16
_NW = _NUM_CORES * _NUM_SUBCORES
_B_PER_W = BATCH // _NW
_CHUNK = 128
_NCHUNK = _B_PER_W // _CHUNK

_mesh = plsc.VectorSubcoreMesh(core_axis_name="c", subcore_axis_name="s")


@functools.partial(
    pl.kernel,
    mesh=_mesh,
    out_type=jax.ShapeDtypeStruct((BATCH, D_AXIOM), jnp.float32),
    compiler_params=pltpu.CompilerParams(
        use_tc_tiling_on_sc=False, needs_layout_passes=False
    ),
    scratch_types=[
        pltpu.VMEM((_NCHUNK, _CHUNK), jnp.int32),
        pltpu.VMEM((_CHUNK, D_PAD), jnp.float32),
        pltpu.VMEM((_CHUNK, D_PAD), jnp.float32),
        pltpu.SemaphoreType.DMA,
        pltpu.SemaphoreType.DMA,
        pltpu.SemaphoreType.DMA,
        pltpu.SemaphoreType.DMA,
    ],
)
def _gather_kernel(idx_hbm, table_hbm, out_hbm,
                   idx_v, rows0, rows1, sg0, sg1, sw0, sw1):
    wid = lax.axis_index("s") * _NUM_CORES + lax.axis_index("c")
    base = wid * _B_PER_W
    pltpu.sync_copy(idx_hbm.at[wid], idx_v)
    rows = (rows0, rows1)
    sgs = (sg0, sg1)
    sws = (sw0, sw1)
    gcp = [pltpu.async_copy(table_hbm.at[idx_v.at[0]], rows0, sg0), None]
    wcp = [None, None]
    for j in range(_NCHUNK):
        cur = j % 2
        nxt = (j + 1) % 2
        if j + 1 < _NCHUNK:
            gcp[nxt] = pltpu.async_copy(
                table_hbm.at[idx_v.at[j + 1]], rows[nxt], sgs[nxt]
            )
        gcp[cur].wait()
        if wcp[cur] is not None:
            wcp[cur].wait()
        wcp[cur] = pltpu.async_copy(
            rows[cur].at[:, pl.ds(0, D_TILE)],
            out_hbm.at[pl.ds(base + j * _CHUNK, _CHUNK), pl.ds(0, D_TILE)],
            sws[cur],
        )
    wcp[0].wait()
    wcp[1].wait()


def kernel(indices, table):
    idx = indices.astype(jnp.int32).reshape(_NW, _NCHUNK, _CHUNK)
    table_pad = jnp.pad(table, ((0, 0), (0, D_PAD - D_AXIOM)))
    return _gather_kernel(idx, table_pad)
